# baseline (speedup 1.0000x reference)
import functools

import jax
import jax.numpy as jnp
from jax import lax
from jax.experimental import pallas as pl
from jax.experimental.pallas import tpu as pltpu
from jax.experimental.pallas import tpu_sc as plsc


def _make_broadcast_copy(B, L, D, dtype):
    info = plsc.get_sparse_core_info()
    num_workers = info.num_cores * info.num_subcores
    assert L % num_workers == 0
    rows_per_w = L // num_workers

    mesh = plsc.VectorSubcoreMesh(core_axis_name="c", subcore_axis_name="s")

    CHUNK = 8
    NBUF = 8
    n_chunks = rows_per_w // CHUNK
    assert n_chunks % NBUF == 0 and n_chunks >= 2 * NBUF

    @functools.partial(
        pl.kernel,
        out_type=jax.ShapeDtypeStruct((B, L, D), dtype),
        mesh=mesh,
        scratch_types=[pltpu.VMEM((NBUF * CHUNK, D), dtype)]
        + [pltpu.SemaphoreType.DMA] * (2 * NBUF),
    )
    def broadcast_copy(table_hbm, out_hbm, bufs, *sems):
        rsems = sems[:NBUF]
        wsems = sems[NBUF:]
        wid = lax.axis_index("s") * info.num_cores + lax.axis_index("c")
        base = wid * rows_per_w

        def buf(k):
            return bufs.at[pl.ds(k * CHUNK, CHUNK)]

        def issue_read(i, k):
            pltpu.async_copy(
                table_hbm.at[pl.ds(base + i * CHUNK, CHUNK)],
                buf(k),
                rsems[k],
            )

        def wait_read(k):
            pltpu.make_async_copy(
                table_hbm.at[pl.ds(base, CHUNK)], buf(k), rsems[k]
            ).wait()

        def issue_writes(i, k):
            for b in range(B):
                pltpu.async_copy(
                    buf(k),
                    out_hbm.at[b, pl.ds(base + i * CHUNK, CHUNK)],
                    wsems[k],
                )

        def wait_writes(k):
            pltpu.make_async_copy(
                table_hbm.at[pl.ds(base, B * CHUNK)],
                bufs.at[pl.ds(0, B * CHUNK)],
                wsems[k],
            ).wait()

        for k in range(NBUF):
            issue_read(k, k)

        def step(g, carry):
            i0 = g * NBUF
            for k in range(NBUF):
                wait_read(k)
                issue_writes(i0 + k, k)
            for k in range(NBUF):
                nxt = i0 + NBUF + k
                wait_writes(k)

                @pl.when(nxt < n_chunks)
                def _issue(nxt=nxt, k=k):
                    issue_read(nxt, k)

            return carry

        lax.fori_loop(0, n_chunks // NBUF, step, 0)

    return broadcast_copy


def kernel(x, embedding_table):
    B, L, D = x.shape
    fn = _make_broadcast_copy(B, L, D, embedding_table.dtype)
    return fn(embedding_table)


# speedup vs baseline: 1.0115x; 1.0115x over previous
import functools

import jax
import jax.numpy as jnp
from jax import lax
from jax.experimental import pallas as pl
from jax.experimental.pallas import tpu as pltpu
from jax.experimental.pallas import tpu_sc as plsc


def _make_broadcast_copy(B, L, D, dtype):
    info = plsc.get_sparse_core_info()
    num_workers = info.num_cores * info.num_subcores
    assert L % num_workers == 0
    rows_per_w = L // num_workers

    mesh = plsc.VectorSubcoreMesh(core_axis_name="c", subcore_axis_name="s")

    CHUNK = 8
    NBUF = 8
    n_chunks = rows_per_w // CHUNK
    assert n_chunks % NBUF == 0 and n_chunks >= 2 * NBUF

    @functools.partial(
        pl.kernel,
        out_type=jax.ShapeDtypeStruct((B, L, D), dtype),
        mesh=mesh,
        scratch_types=[pltpu.VMEM((NBUF * CHUNK, D), dtype)]
        + [pltpu.SemaphoreType.DMA] * (2 * NBUF),
    )
    def broadcast_copy(table_hbm, out_hbm, bufs, *sems):
        rsems = sems[:NBUF]
        wsems = sems[NBUF:]
        wid = lax.axis_index("c") * info.num_subcores + lax.axis_index("s")
        base = wid * rows_per_w

        def buf(k):
            return bufs.at[pl.ds(k * CHUNK, CHUNK)]

        def issue_read(i, k):
            pltpu.async_copy(
                table_hbm.at[pl.ds(base + i * CHUNK, CHUNK)],
                buf(k),
                rsems[k],
            )

        def wait_read(k):
            pltpu.make_async_copy(
                table_hbm.at[pl.ds(base, CHUNK)], buf(k), rsems[k]
            ).wait()

        def issue_writes(i, k):
            for b in range(B):
                pltpu.async_copy(
                    buf(k),
                    out_hbm.at[b, pl.ds(base + i * CHUNK, CHUNK)],
                    wsems[k],
                )

        def wait_writes(k):
            pltpu.make_async_copy(
                table_hbm.at[pl.ds(base, B * CHUNK)],
                bufs.at[pl.ds(0, B * CHUNK)],
                wsems[k],
            ).wait()

        for k in range(NBUF):
            issue_read(k, k)

        def step(g, carry):
            i0 = g * NBUF
            for k in range(NBUF):
                wait_read(k)
                issue_writes(i0 + k, k)
            for k in range(NBUF):
                nxt = i0 + NBUF + k
                wait_writes(k)

                @pl.when(nxt < n_chunks)
                def _issue(nxt=nxt, k=k):
                    issue_read(nxt, k)

            return carry

        lax.fori_loop(0, n_chunks // NBUF, step, 0)

    return broadcast_copy


def kernel(x, embedding_table):
    B, L, D = x.shape
    fn = _make_broadcast_copy(B, L, D, embedding_table.dtype)
    return fn(embedding_table)
